# baseline (speedup 1.0000x reference)
import functools

import jax
import jax.numpy as jnp
from jax import lax
from jax.experimental import pallas as pl
from jax.experimental.pallas import tpu as pltpu
from jax.experimental.pallas import tpu_sc as plsc

NC = 2
NS = 16
BLK = 2048
MAGIC = 0x5F3759DF


def _sc_aggregate(x, edge_index, n, f, npad, e, ch):
    FP = 8
    np16 = npad // NS
    xc = np16 // 16
    n_node_chunks = np16 // xc
    ec = e // (NC * NS)
    deg_per_tile = e // NS
    n_deg_chunks = deg_per_tile // ch
    n_edge_chunks = ec // ch
    flat = xc * FP

    mesh = plsc.VectorSubcoreMesh(core_axis_name="c", subcore_axis_name="s")

    @functools.partial(
        pl.kernel,
        out_type=jax.ShapeDtypeStruct((NC * npad, FP), jnp.float32),
        mesh=mesh,
        compiler_params=pltpu.CompilerParams(
            needs_layout_passes=False, use_tc_tiling_on_sc=False),
        scratch_types=[
            pltpu.VMEM_SHARED((npad, FP), jnp.float32),
            pltpu.VMEM_SHARED((npad, FP), jnp.float32),
            pltpu.VMEM_SHARED((npad,), jnp.float32),
            pltpu.VMEM((ch,), jnp.float32),
            pltpu.VMEM((np16,), jnp.float32),
            pltpu.VMEM((xc * f,), jnp.float32),
            pltpu.VMEM((xc, FP), jnp.float32),
            pltpu.VMEM((ch,), jnp.int32),
            pltpu.VMEM((ch,), jnp.int32),
            pltpu.VMEM((ch, FP), jnp.float32),
        ],
    )
    def sc_kernel(x_hbm, src_hbm, dst_hbm, agg_hbm,
                  xs_sh, agg_sh, deg_sh,
                  ones_v, dinv_v, x6_v, x8_v, sidx_v, didx_v, rows_v):
        c = lax.axis_index("c")
        s = lax.axis_index("s")
        t0 = s * np16
        lane = lax.iota(jnp.int32, 16)
        fvec = jnp.full((16,), jnp.int32(f), jnp.int32)

        def fill_ones(i, carry):
            ones_v[pl.ds(i * 16, 16)] = jnp.full((16,), 1.0, jnp.float32)
            return carry
        lax.fori_loop(0, ch // 16, fill_ones, 0)
        for q in range(np16 // xc):
            pltpu.sync_copy(ones_v.at[pl.ds(0, xc)],
                            deg_sh.at[pl.ds(t0 + q * xc, xc)])
        plsc.subcore_barrier()

        def deg_step(i, carry):
            off = s * deg_per_tile + i * ch
            pltpu.sync_copy(dst_hbm.at[pl.ds(off, ch)], didx_v)
            pltpu.sync_copy(ones_v.at[pl.ds(0, ch)], deg_sh.at[didx_v],
                            add=True)
            return carry
        lax.fori_loop(0, n_deg_chunks, deg_step, 0)
        plsc.subcore_barrier()

        pltpu.sync_copy(deg_sh.at[pl.ds(t0, np16)], dinv_v)
        magic = jnp.full((16,), MAGIC, jnp.int32)
        def rsq_step(i, carry):
            y = dinv_v[pl.ds(i * 16, 16)]
            bi = magic - lax.shift_right_arithmetic(
                plsc.bitcast(y, jnp.int32), 1)
            z = plsc.bitcast(bi, jnp.float32)
            z = z * (1.5 - 0.5 * y * z * z)
            z = z * (1.5 - 0.5 * y * z * z)
            z = z * (1.5 - 0.5 * y * z * z)
            dinv_v[pl.ds(i * 16, 16)] = z
            return carry
        lax.fori_loop(0, np16 // 16, rsq_step, 0)

        zero16 = jnp.zeros((16,), jnp.float32)
        rr8 = lax.shift_right_arithmetic(lane, 3)
        cc8 = lane & 7

        def node_chunk(ci, carry):
            go = t0 + ci * xc
            lb = ci * xc

            @pl.when(go + xc <= n)
            def _():
                pltpu.sync_copy(x_hbm.at[pl.ds(go * f, xc * f)], x6_v)

            @pl.when(go + xc > n)
            def _():
                def zstep(j, carry2):
                    plsc.store_scatter(x6_v, [j * 16 + lane], zero16)
                    return carry2
                lax.fori_loop(0, (xc * f) // 16, zstep, 0)
                tail = n - (n // xc) * xc
                if tail:
                    pltpu.sync_copy(
                        x_hbm.at[pl.ds((n // xc) * xc * f, tail * f)],
                        x6_v.at[pl.ds(0, tail * f)])

            def xs_step(j, carry2):
                row = rr8 + 2 * j
                d16 = plsc.load_gather(dinv_v, [lb + row])
                v16 = plsc.load_gather(
                    x6_v, [row * f + jnp.minimum(cc8, jnp.int32(f - 1))])
                val = jnp.where(cc8 < f, v16 * d16, 0.0)
                plsc.store_scatter(x8_v, [row, cc8], val)
                return carry2
            lax.fori_loop(0, flat // 16, xs_step, 0)
            pltpu.sync_copy(x8_v, xs_sh.at[pl.ds(go, xc)])

            @pl.when(c == 0)
            def _():
                pltpu.sync_copy(x8_v, agg_sh.at[pl.ds(go, xc)])
            return carry
        lax.fori_loop(0, n_node_chunks, node_chunk, 0)

        @pl.when(c == 1)
        def _():
            def z8step(j, carry2):
                plsc.store_scatter(x8_v, [rr8 + 2 * j, cc8], zero16)
                return carry2
            lax.fori_loop(0, flat // 16, z8step, 0)
            def zc_step(ci, carry2):
                pltpu.sync_copy(x8_v, agg_sh.at[pl.ds(t0 + ci * xc, xc)])
                return carry2
            lax.fori_loop(0, n_node_chunks, zc_step, 0)
        plsc.subcore_barrier()

        wid = c * NS + s
        def edge_step(i, carry):
            off = wid * ec + i * ch
            pltpu.sync_copy(src_hbm.at[pl.ds(off, ch)], sidx_v)
            pltpu.sync_copy(dst_hbm.at[pl.ds(off, ch)], didx_v)
            pltpu.sync_copy(xs_sh.at[sidx_v], rows_v)
            pltpu.sync_copy(rows_v, agg_sh.at[didx_v], add=True)
            return carry
        lax.fori_loop(0, n_edge_chunks, edge_step, 0)
        plsc.subcore_barrier()

        def out_chunk(ci, carry):
            go = t0 + ci * xc
            lb = ci * xc
            pltpu.sync_copy(agg_sh.at[pl.ds(go, xc)], x8_v)
            def sc_step(j, carry2):
                row = rr8 + 2 * j
                d16 = plsc.load_gather(dinv_v, [lb + row])
                v16 = plsc.load_gather(x8_v, [row, cc8])
                plsc.store_scatter(x8_v, [row, cc8], v16 * d16)
                return carry2
            lax.fori_loop(0, flat // 16, sc_step, 0)
            pltpu.sync_copy(x8_v, agg_hbm.at[pl.ds(c * npad + go, xc)])
            return carry
        lax.fori_loop(0, n_node_chunks, out_chunk, 0)

    return sc_kernel(x.reshape(-1), edge_index[0], edge_index[1])


def _tc_dense(agg, bn2, W1p, b1r, W2, b2r, W3, b3r, n, npad, g, h, a):
    nb = npad // BLK
    fp = W1p.shape[0]

    def tc_body(a0_ref, a1_ref, bn_ref,
                w1_ref, b1_ref, w2_ref, b2_ref, w3_ref, b3_ref,
                out_ref, sums_ref, cnt_ref):
        i = pl.program_id(0)

        @pl.when(i == 0)
        def _():
            sums_ref[...] = jnp.zeros_like(sums_ref)
            cnt_ref[...] = jnp.zeros_like(cnt_ref)

        node = a0_ref[...] + a1_ref[...]
        hblk = jnp.maximum(
            jnp.dot(node, w1_ref[...], preferred_element_type=jnp.float32)
            + b1_ref[...], 0.0)
        ids = bn_ref[...]
        valid = (lax.broadcasted_iota(jnp.int32, (1, BLK), 1)
                 + i * BLK) < n
        onehot = ((lax.broadcasted_iota(jnp.int32, (g, BLK), 0) == ids)
                  & valid).astype(jnp.float32)
        sums_ref[...] += jnp.dot(onehot, hblk,
                                 preferred_element_type=jnp.float32)
        cnt_ref[...] += jnp.sum(onehot, axis=1, keepdims=True)

        @pl.when(i == nb - 1)
        def _():
            mean = sums_ref[...] / jnp.maximum(cnt_ref[...], 1.0)
            h2 = jnp.maximum(
                jnp.dot(mean, w2_ref[...], preferred_element_type=jnp.float32)
                + b2_ref[...], 0.0)
            logits = jnp.dot(h2, w3_ref[...],
                             preferred_element_type=jnp.float32) + b3_ref[...]
            m = jnp.max(logits, axis=1, keepdims=True)
            lse = jnp.log(jnp.sum(jnp.exp(logits - m), axis=1,
                                  keepdims=True)) + m
            out_ref[...] = logits - lse

    full = lambda shape: pl.BlockSpec(shape, lambda i: (0,) * len(shape))
    return pl.pallas_call(
        tc_body,
        grid=(nb,),
        in_specs=[
            pl.BlockSpec((BLK, fp), lambda i: (i, 0)),
            pl.BlockSpec((BLK, fp), lambda i: (i + nb, 0)),
            pl.BlockSpec((1, BLK), lambda i: (0, i)),
            full((fp, h)), full((1, h)),
            full((h, h)), full((1, h)),
            full((h, a)), full((1, a)),
        ],
        out_specs=pl.BlockSpec((g, a), lambda i: (0, 0)),
        out_shape=jax.ShapeDtypeStruct((g, a), jnp.float32),
        scratch_shapes=[
            pltpu.VMEM((g, h), jnp.float32),
            pltpu.VMEM((g, 1), jnp.float32),
        ],
    )(agg, agg, bn2, W1p, b1r, W2, b2r, W3, b3r)


def kernel(x, edge_index, batch_number, W1, b1, W2, b2, W3, b3):
    n, f = x.shape
    e = edge_index.shape[1]
    h = W1.shape[1]
    a = W3.shape[1]
    g = 256

    npad = -(-n // BLK) * BLK
    e32 = e // (NC * NS)
    ch = next(c for c in range(1024, 0, -8) if e32 % c == 0)

    agg = _sc_aggregate(x, edge_index, n, f, npad, e, ch)
    W1p = jnp.zeros((8, h), jnp.float32).at[:f].set(W1)
    return _tc_dense(agg, batch_number.reshape(1, n), W1p,
                     b1.reshape(1, h), W2, b2.reshape(1, h),
                     W3, b3.reshape(1, a), n, npad, g, h, a)


# speedup vs baseline: 1.2515x; 1.2515x over previous
import functools

import jax
import jax.numpy as jnp
from jax import lax
from jax.experimental import pallas as pl
from jax.experimental.pallas import tpu as pltpu
from jax.experimental.pallas import tpu_sc as plsc

NC = 2
NS = 16
BLK = 2048
MAGIC = 0x5F3759DF


def _sc_aggregate(x, edge_index, n, f, npad, e, ch):
    FP = 8
    np16 = npad // NS
    xc = np16 // 16
    n_node_chunks = np16 // xc
    ec = e // (NC * NS)
    deg_per_tile = e // NS
    n_deg_chunks = deg_per_tile // ch
    n_edge_chunks = ec // ch
    flat = xc * FP

    mesh = plsc.VectorSubcoreMesh(core_axis_name="c", subcore_axis_name="s")

    @functools.partial(
        pl.kernel,
        out_type=jax.ShapeDtypeStruct((NC * npad // 128, FP, 128),
                                      jnp.float32),
        mesh=mesh,
        compiler_params=pltpu.CompilerParams(
            needs_layout_passes=False, use_tc_tiling_on_sc=False),
        scratch_types=[
            pltpu.VMEM_SHARED((npad, FP), jnp.float32),
            pltpu.VMEM_SHARED((npad, FP), jnp.float32),
            pltpu.VMEM_SHARED((npad,), jnp.float32),
            pltpu.VMEM((ch,), jnp.float32),
            pltpu.VMEM((np16,), jnp.float32),
            pltpu.VMEM((xc * f,), jnp.float32),
            pltpu.VMEM((xc, FP), jnp.float32),
            pltpu.VMEM((FP, 128), jnp.float32),
            pltpu.VMEM((ch,), jnp.int32),
            pltpu.VMEM((ch,), jnp.int32),
            pltpu.VMEM((ch, FP), jnp.float32),
        ],
    )
    def sc_kernel(x_hbm, edge_hbm, agg_hbm,
                  xs_sh, agg_sh, deg_sh,
                  ones_v, dinv_v, x6_v, x8_v, xt_v, sidx_v, didx_v, rows_v):
        c = lax.axis_index("c")
        s = lax.axis_index("s")
        t0 = s * np16
        lane = lax.iota(jnp.int32, 16)
        fvec = jnp.full((16,), jnp.int32(f), jnp.int32)

        def fill_ones(i, carry):
            ones_v[pl.ds(i * 16, 16)] = jnp.full((16,), 1.0, jnp.float32)
            return carry
        lax.fori_loop(0, ch // 16, fill_ones, 0)
        for q in range(np16 // xc):
            pltpu.sync_copy(ones_v.at[pl.ds(0, xc)],
                            deg_sh.at[pl.ds(t0 + q * xc, xc)])
        plsc.subcore_barrier()

        def deg_step(i, carry):
            off = s * deg_per_tile + i * ch
            pltpu.sync_copy(edge_hbm.at[1, pl.ds(off, ch)], didx_v)
            pltpu.sync_copy(ones_v.at[pl.ds(0, ch)], deg_sh.at[didx_v],
                            add=True)
            return carry
        lax.fori_loop(0, n_deg_chunks, deg_step, 0)
        plsc.subcore_barrier()

        pltpu.sync_copy(deg_sh.at[pl.ds(t0, np16)], dinv_v)
        magic = jnp.full((16,), MAGIC, jnp.int32)
        def rsq_step(i, carry):
            y = dinv_v[pl.ds(i * 16, 16)]
            bi = magic - lax.shift_right_arithmetic(
                plsc.bitcast(y, jnp.int32), 1)
            z = plsc.bitcast(bi, jnp.float32)
            z = z * (1.5 - 0.5 * y * z * z)
            z = z * (1.5 - 0.5 * y * z * z)
            z = z * (1.5 - 0.5 * y * z * z)
            dinv_v[pl.ds(i * 16, 16)] = z
            return carry
        lax.fori_loop(0, np16 // 16, rsq_step, 0)

        zero16 = jnp.zeros((16,), jnp.float32)
        rr8 = lax.shift_right_arithmetic(lane, 3)
        cc8 = lane & 7

        def node_chunk(ci, carry):
            go = t0 + ci * xc
            lb = ci * xc

            @pl.when(go + xc <= n)
            def _():
                pltpu.sync_copy(x_hbm.at[pl.ds(go * f, xc * f)], x6_v)

            @pl.when(go + xc > n)
            def _():
                def zstep(j, carry2):
                    plsc.store_scatter(x6_v, [j * 16 + lane], zero16)
                    return carry2
                lax.fori_loop(0, (xc * f) // 16, zstep, 0)
                tail = n - (n // xc) * xc
                if tail:
                    pltpu.sync_copy(
                        x_hbm.at[pl.ds((n // xc) * xc * f, tail * f)],
                        x6_v.at[pl.ds(0, tail * f)])

            def xs_step(j, carry2):
                row = rr8 + 2 * j
                d16 = plsc.load_gather(dinv_v, [lb + row])
                v16 = plsc.load_gather(
                    x6_v, [row * f + jnp.minimum(cc8, jnp.int32(f - 1))])
                val = jnp.where(cc8 < f, v16 * d16, 0.0)
                plsc.store_scatter(x8_v, [row, cc8], val)
                return carry2
            lax.fori_loop(0, flat // 16, xs_step, 0)
            pltpu.sync_copy(x8_v, xs_sh.at[pl.ds(go, xc)])

            @pl.when(c == 0)
            def _():
                pltpu.sync_copy(x8_v, agg_sh.at[pl.ds(go, xc)])
            return carry
        lax.fori_loop(0, n_node_chunks, node_chunk, 0)

        @pl.when(c == 1)
        def _():
            def z8step(j, carry2):
                plsc.store_scatter(x8_v, [rr8 + 2 * j, cc8], zero16)
                return carry2
            lax.fori_loop(0, flat // 16, z8step, 0)
            def zc_step(ci, carry2):
                pltpu.sync_copy(x8_v, agg_sh.at[pl.ds(t0 + ci * xc, xc)])
                return carry2
            lax.fori_loop(0, n_node_chunks, zc_step, 0)
        plsc.subcore_barrier()

        wid = c * NS + s
        def edge_step(i, carry):
            off = wid * ec + i * ch
            pltpu.sync_copy(edge_hbm.at[0, pl.ds(off, ch)], sidx_v)
            pltpu.sync_copy(edge_hbm.at[1, pl.ds(off, ch)], didx_v)
            pltpu.sync_copy(xs_sh.at[sidx_v], rows_v)
            pltpu.sync_copy(rows_v, agg_sh.at[didx_v], add=True)
            return carry
        lax.fori_loop(0, n_edge_chunks, edge_step, 0)
        plsc.subcore_barrier()

        n_out_tiles = np16 // 128
        def out_group(gk, carry):
            go = t0 + gk * 128
            pltpu.sync_copy(agg_sh.at[pl.ds(go, 128)],
                            x8_v.at[pl.ds(0, 128)])
            def sc_step(j, carry2):
                row = rr8 + 2 * j
                d16 = plsc.load_gather(dinv_v, [gk * 128 + row])
                v16 = plsc.load_gather(x8_v, [row, cc8])
                plsc.store_scatter(xt_v, [cc8, row], v16 * d16)
                return carry2
            lax.fori_loop(0, 64, sc_step, 0)
            pltpu.sync_copy(
                xt_v, agg_hbm.at[c * (npad // 128) + s * n_out_tiles + gk])
            return carry
        lax.fori_loop(0, n_out_tiles, out_group, 0)

    return sc_kernel(x.reshape(-1), edge_index)


def _tc_dense(agg3, bn2, W1p, b1r, W2, b2r, W3, b3r, n, npad, g, h, a):
    nb = npad // BLK
    tb = BLK // 128
    fp = W1p.shape[0]

    def tc_body(a0_ref, a1_ref, bn_ref,
                w1_ref, b1_ref, w2_ref, b2_ref, w3_ref, b3_ref,
                out_ref, sums_ref, cnt_ref):
        i = pl.program_id(0)

        @pl.when(i == 0)
        def _():
            sums_ref[...] = jnp.zeros_like(sums_ref)
            cnt_ref[...] = jnp.zeros_like(cnt_ref)

        node3 = a0_ref[...] + a1_ref[...]
        h3 = jnp.maximum(
            lax.dot_general(node3, w1_ref[...], (((1,), (0,)), ((), ())),
                            preferred_element_type=jnp.float32)
            + b1_ref[...][None], 0.0)
        ids3 = bn_ref[...][:, None, :]
        onehot3 = (lax.broadcasted_iota(jnp.int32, (tb, g, 128), 1)
                   == ids3).astype(jnp.float32)
        part = lax.dot_general(onehot3, h3, (((2,), (1,)), ((0,), (0,))),
                               preferred_element_type=jnp.float32)
        sums_ref[...] += jnp.sum(part, axis=0)
        cnt_ref[...] += jnp.sum(onehot3, axis=(0, 2))[None]

        @pl.when(i == nb - 1)
        def _():
            cnt = lax.transpose(cnt_ref[...], (1, 0))
            mean = sums_ref[...] / jnp.maximum(cnt, 1.0)
            h2 = jnp.maximum(
                jnp.dot(mean, w2_ref[...], preferred_element_type=jnp.float32)
                + b2_ref[...], 0.0)
            logits = jnp.dot(h2, w3_ref[...],
                             preferred_element_type=jnp.float32) + b3_ref[...]
            m = jnp.max(logits, axis=1, keepdims=True)
            lse = jnp.log(jnp.sum(jnp.exp(logits - m), axis=1,
                                  keepdims=True)) + m
            out_ref[...] = logits - lse

    full = lambda shape: pl.BlockSpec(shape, lambda i: (0,) * len(shape))
    return pl.pallas_call(
        tc_body,
        grid=(nb,),
        in_specs=[
            pl.BlockSpec((tb, fp, 128), lambda i: (i, 0, 0)),
            pl.BlockSpec((tb, fp, 128), lambda i: (i + nb, 0, 0)),
            pl.BlockSpec((tb, 128), lambda i: (i, 0)),
            full((fp, h)), full((1, h)),
            full((h, h)), full((1, h)),
            full((h, a)), full((1, a)),
        ],
        out_specs=pl.BlockSpec((g, a), lambda i: (0, 0)),
        out_shape=jax.ShapeDtypeStruct((g, a), jnp.float32),
        scratch_shapes=[
            pltpu.VMEM((g, h), jnp.float32),
            pltpu.VMEM((1, g), jnp.float32),
        ],
    )(agg3, agg3, bn2, W1p, b1r, W2, b2r, W3, b3r)


def kernel(x, edge_index, batch_number, W1, b1, W2, b2, W3, b3):
    n, f = x.shape
    e = edge_index.shape[1]
    h = W1.shape[1]
    a = W3.shape[1]
    g = 256

    npad = -(-n // BLK) * BLK
    e32 = e // (NC * NS)
    ch = next(c for c in range(1024, 0, -8) if e32 % c == 0)

    agg3 = _sc_aggregate(x, edge_index, n, f, npad, e, ch)
    W1p = jnp.zeros((8, h), jnp.float32).at[:f].set(W1)
    bn2 = jnp.pad(batch_number, (0, npad - n),
                  constant_values=g).reshape(npad // 128, 128)
    return _tc_dense(agg3, bn2, W1p,
                     b1.reshape(1, h), W2, b2.reshape(1, h),
                     W3, b3.reshape(1, a), n, npad, g, h, a)


# speedup vs baseline: 1.3271x; 1.0604x over previous
import functools

import jax
import jax.numpy as jnp
from jax import lax
from jax.experimental import pallas as pl
from jax.experimental.pallas import tpu as pltpu
from jax.experimental.pallas import tpu_sc as plsc

NC = 2
NS = 16
BLK = 2048
MAGIC = 0x5F3759DF


def _sc_aggregate(x, edge_index, n, f, npad, e, ch):
    FP = 8
    np16 = npad // NS
    xc = np16 // 16
    n_node_chunks = np16 // xc
    ec = e // (NC * NS)
    deg_per_tile = e // NS
    n_deg_chunks = deg_per_tile // ch
    n_edge_chunks = ec // ch
    flat = xc * FP

    mesh = plsc.VectorSubcoreMesh(core_axis_name="c", subcore_axis_name="s")

    @functools.partial(
        pl.kernel,
        out_type=jax.ShapeDtypeStruct((NC * npad // 128, FP, 128),
                                      jnp.float32),
        mesh=mesh,
        compiler_params=pltpu.CompilerParams(
            needs_layout_passes=False, use_tc_tiling_on_sc=False),
        scratch_types=[
            pltpu.VMEM_SHARED((npad, FP), jnp.float32),
            pltpu.VMEM_SHARED((npad, FP), jnp.float32),
            pltpu.VMEM_SHARED((npad,), jnp.float32),
            pltpu.VMEM((ch,), jnp.float32),
            pltpu.VMEM((np16,), jnp.float32),
            pltpu.VMEM((xc * f,), jnp.float32),
            pltpu.VMEM((xc, FP), jnp.float32),
            pltpu.VMEM((FP, 128), jnp.float32),
            pltpu.VMEM((ch,), jnp.int32),
            pltpu.VMEM((ch,), jnp.int32),
            pltpu.VMEM((ch, FP), jnp.float32),
            pltpu.SemaphoreType.DMA,
            pltpu.SemaphoreType.DMA,
        ],
    )
    def sc_kernel(x_hbm, edge_hbm, agg_hbm,
                  xs_sh, agg_sh, deg_sh,
                  ones_v, dinv_v, x6_v, x8_v, xt_v, sidx_v, didx_v, rows_v,
                  sem1, sem2):
        c = lax.axis_index("c")
        s = lax.axis_index("s")
        t0 = s * np16
        lane = lax.iota(jnp.int32, 16)
        fvec = jnp.full((16,), jnp.int32(f), jnp.int32)

        def fill_ones(i, carry):
            ones_v[pl.ds(i * 16, 16)] = jnp.full((16,), 1.0, jnp.float32)
            return carry
        lax.fori_loop(0, ch // 16, fill_ones, 0)
        for q in range(np16 // xc):
            pltpu.sync_copy(ones_v.at[pl.ds(0, xc)],
                            deg_sh.at[pl.ds(t0 + q * xc, xc)])
        plsc.subcore_barrier()

        def deg_step(i, carry):
            off = s * deg_per_tile + i * ch
            pltpu.sync_copy(edge_hbm.at[1, pl.ds(off, ch)], didx_v)
            pltpu.sync_copy(ones_v.at[pl.ds(0, ch)], deg_sh.at[didx_v],
                            add=True)
            return carry
        lax.fori_loop(0, n_deg_chunks, deg_step, 0)
        plsc.subcore_barrier()

        pltpu.sync_copy(deg_sh.at[pl.ds(t0, np16)], dinv_v)
        magic = jnp.full((16,), MAGIC, jnp.int32)
        def rsq_step(i, carry):
            y = dinv_v[pl.ds(i * 16, 16)]
            bi = magic - lax.shift_right_arithmetic(
                plsc.bitcast(y, jnp.int32), 1)
            z = plsc.bitcast(bi, jnp.float32)
            z = z * (1.5 - 0.5 * y * z * z)
            z = z * (1.5 - 0.5 * y * z * z)
            z = z * (1.5 - 0.5 * y * z * z)
            dinv_v[pl.ds(i * 16, 16)] = z
            return carry
        lax.fori_loop(0, np16 // 16, rsq_step, 0)

        zero16 = jnp.zeros((16,), jnp.float32)
        rr8 = lax.shift_right_arithmetic(lane, 3)
        cc8 = lane & 7

        def node_chunk(ci, carry):
            go = t0 + ci * xc
            lb = ci * xc

            @pl.when(go + xc <= n)
            def _():
                pltpu.sync_copy(x_hbm.at[pl.ds(go * f, xc * f)], x6_v)

            @pl.when(go + xc > n)
            def _():
                def zstep(j, carry2):
                    plsc.store_scatter(x6_v, [j * 16 + lane], zero16)
                    return carry2
                lax.fori_loop(0, (xc * f) // 16, zstep, 0)
                tail = n - (n // xc) * xc
                if tail:
                    pltpu.sync_copy(
                        x_hbm.at[pl.ds((n // xc) * xc * f, tail * f)],
                        x6_v.at[pl.ds(0, tail * f)])

            def xs_step(j, carry2):
                row = rr8 + 2 * j
                d16 = plsc.load_gather(dinv_v, [lb + row])
                v16 = plsc.load_gather(
                    x6_v, [row * f + jnp.minimum(cc8, jnp.int32(f - 1))])
                val = jnp.where(cc8 < f, v16 * d16, 0.0)
                plsc.store_scatter(x8_v, [row, cc8], val)
                return carry2
            lax.fori_loop(0, flat // 16, xs_step, 0)
            pltpu.sync_copy(x8_v, xs_sh.at[pl.ds(go, xc)])

            @pl.when(c == 0)
            def _():
                pltpu.sync_copy(x8_v, agg_sh.at[pl.ds(go, xc)])
            return carry
        lax.fori_loop(0, n_node_chunks, node_chunk, 0)

        @pl.when(c == 1)
        def _():
            def z8step(j, carry2):
                plsc.store_scatter(x8_v, [rr8 + 2 * j, cc8], zero16)
                return carry2
            lax.fori_loop(0, flat // 16, z8step, 0)
            def zc_step(ci, carry2):
                pltpu.sync_copy(x8_v, agg_sh.at[pl.ds(t0 + ci * xc, xc)])
                return carry2
            lax.fori_loop(0, n_node_chunks, zc_step, 0)
        plsc.subcore_barrier()

        wid = c * NS + s
        def edge_step(i, carry):
            off = wid * ec + i * ch
            cp1 = pltpu.async_copy(edge_hbm.at[0, pl.ds(off, ch)], sidx_v,
                                   sem1)
            cp2 = pltpu.async_copy(edge_hbm.at[1, pl.ds(off, ch)], didx_v,
                                   sem2)
            cp1.wait()
            cp2.wait()
            pltpu.sync_copy(xs_sh.at[sidx_v], rows_v)
            pltpu.sync_copy(rows_v, agg_sh.at[didx_v], add=True)
            return carry
        lax.fori_loop(0, n_edge_chunks, edge_step, 0)
        plsc.subcore_barrier()

        n_out_tiles = np16 // 128
        def out_group(gk, carry):
            go = t0 + gk * 128
            pltpu.sync_copy(agg_sh.at[pl.ds(go, 128)],
                            x8_v.at[pl.ds(0, 128)])
            def sc_step(j, carry2):
                row = rr8 + 2 * j
                d16 = plsc.load_gather(dinv_v, [gk * 128 + row])
                v16 = plsc.load_gather(x8_v, [row, cc8])
                plsc.store_scatter(xt_v, [cc8, row], v16 * d16)
                return carry2
            lax.fori_loop(0, 64, sc_step, 0)
            pltpu.sync_copy(
                xt_v, agg_hbm.at[c * (npad // 128) + s * n_out_tiles + gk])
            return carry
        lax.fori_loop(0, n_out_tiles, out_group, 0)

    return sc_kernel(x.reshape(-1), edge_index)


def _tc_dense(agg3, bn2, W1p, b1r, W2, b2r, W3, b3r, n, npad, g, h, a):
    nb = npad // BLK
    tb = BLK // 128
    fp = W1p.shape[0]

    def tc_body(a0_ref, a1_ref, bn_ref,
                w1_ref, b1_ref, w2_ref, b2_ref, w3_ref, b3_ref,
                out_ref, sums_ref, cnt_ref):
        i = pl.program_id(0)

        @pl.when(i == 0)
        def _():
            sums_ref[...] = jnp.zeros_like(sums_ref)
            cnt_ref[...] = jnp.zeros_like(cnt_ref)

        node3 = a0_ref[...] + a1_ref[...]
        h3 = jnp.maximum(
            lax.dot_general(node3, w1_ref[...], (((1,), (0,)), ((), ())),
                            preferred_element_type=jnp.float32)
            + b1_ref[...][None], 0.0)
        ids3 = bn_ref[...][:, None, :]
        onehot3 = (lax.broadcasted_iota(jnp.int32, (tb, g, 128), 1)
                   == ids3).astype(jnp.float32)
        part = lax.dot_general(onehot3, h3, (((2,), (1,)), ((0,), (0,))),
                               preferred_element_type=jnp.float32)
        sums_ref[...] += jnp.sum(part, axis=0)
        cnt_ref[...] += jnp.sum(onehot3, axis=(0, 2))[None]

        @pl.when(i == nb - 1)
        def _():
            cnt = lax.transpose(cnt_ref[...], (1, 0))
            mean = sums_ref[...] / jnp.maximum(cnt, 1.0)
            h2 = jnp.maximum(
                jnp.dot(mean, w2_ref[...], preferred_element_type=jnp.float32)
                + b2_ref[...], 0.0)
            logits = jnp.dot(h2, w3_ref[...],
                             preferred_element_type=jnp.float32) + b3_ref[...]
            m = jnp.max(logits, axis=1, keepdims=True)
            lse = jnp.log(jnp.sum(jnp.exp(logits - m), axis=1,
                                  keepdims=True)) + m
            out_ref[...] = logits - lse

    full = lambda shape: pl.BlockSpec(shape, lambda i: (0,) * len(shape))
    return pl.pallas_call(
        tc_body,
        grid=(nb,),
        in_specs=[
            pl.BlockSpec((tb, fp, 128), lambda i: (i, 0, 0)),
            pl.BlockSpec((tb, fp, 128), lambda i: (i + nb, 0, 0)),
            pl.BlockSpec((tb, 128), lambda i: (i, 0)),
            full((fp, h)), full((1, h)),
            full((h, h)), full((1, h)),
            full((h, a)), full((1, a)),
        ],
        out_specs=pl.BlockSpec((g, a), lambda i: (0, 0)),
        out_shape=jax.ShapeDtypeStruct((g, a), jnp.float32),
        scratch_shapes=[
            pltpu.VMEM((g, h), jnp.float32),
            pltpu.VMEM((1, g), jnp.float32),
        ],
    )(agg3, agg3, bn2, W1p, b1r, W2, b2r, W3, b3r)


def kernel(x, edge_index, batch_number, W1, b1, W2, b2, W3, b3):
    n, f = x.shape
    e = edge_index.shape[1]
    h = W1.shape[1]
    a = W3.shape[1]
    g = 256

    npad = -(-n // BLK) * BLK
    e32 = e // (NC * NS)
    ch = next(c for c in range(1024, 0, -8) if e32 % c == 0)

    agg3 = _sc_aggregate(x, edge_index, n, f, npad, e, ch)
    W1p = jnp.zeros((8, h), jnp.float32).at[:f].set(W1)
    bn2 = jnp.pad(batch_number, (0, npad - n),
                  constant_values=g).reshape(npad // 128, 128)
    return _tc_dense(agg3, bn2, W1p,
                     b1.reshape(1, h), W2, b2.reshape(1, h),
                     W3, b3.reshape(1, a), n, npad, g, h, a)
